# Initial kernel scaffold; baseline (speedup 1.0000x reference)
import jax
import jax.numpy as jnp
from jax.experimental import pallas as pl


def kernel(eeg_input_ids, ecg_input_ids, eeg_table, ecg_table):
    raise NotImplementedError("write your pallas kernel here")


# speedup vs baseline: 3.3169x; 3.3169x over previous
import functools

import jax
import jax.numpy as jnp
from jax import lax
from jax.experimental import pallas as pl
from jax.experimental.pallas import tpu as pltpu
from jax.experimental.pallas import tpu_sc as plsc

B = 4096
L = 50
HID = 128
TOTAL = B * L
NW = 32
PER_W = TOTAL // NW
CHUNK = 128
NCHUNK = PER_W // CHUNK


def _body(eeg_tab, ecg_tab, eeg_idx, ecg_idx, eeg_out, ecg_out,
          idx_v, buf0, buf1, sem0, sem1):
    wid = lax.axis_index("c") * 16 + lax.axis_index("s")
    row_base = wid * PER_W

    for (tab, idx_hbm, out_hbm, ivm) in (
        (eeg_tab, eeg_idx, eeg_out, 0),
        (ecg_tab, ecg_idx, ecg_out, 1),
    ):
        pltpu.sync_copy(idx_hbm.at[pl.ds(row_base, PER_W)], idx_v.at[ivm])

        pltpu.async_copy(tab.at[idx_v.at[ivm, pl.ds(0, CHUNK)]], buf0, sem0)

        def step(jj, _):
            for b, (buf, sem, obuf, osem) in enumerate(
                ((buf0, sem0, buf1, sem1), (buf1, sem1, buf0, sem0))):
                cj = jj * 2 + b
                nj = cj + 1
                pltpu.make_async_copy(
                    tab.at[idx_v.at[ivm, pl.ds(cj * CHUNK, CHUNK)]], buf,
                    sem).wait()

                @pl.when(nj < NCHUNK)
                def _fire():
                    pltpu.async_copy(
                        tab.at[idx_v.at[ivm, pl.ds(nj * CHUNK, CHUNK)]],
                        obuf, osem)

                pltpu.sync_copy(
                    buf, out_hbm.at[pl.ds(row_base + cj * CHUNK, CHUNK)])
            return _

        lax.fori_loop(0, NCHUNK // 2, step, None)


@functools.partial(jax.jit, static_argnums=())
def kernel(eeg_input_ids, ecg_input_ids, eeg_table, ecg_table):
    eeg_idx = eeg_input_ids.reshape(TOTAL).astype(jnp.int32)
    ecg_idx = ecg_input_ids.reshape(TOTAL).astype(jnp.int32)

    mesh = plsc.VectorSubcoreMesh(core_axis_name="c", subcore_axis_name="s")
    run = pl.kernel(
        _body,
        mesh=mesh,
        out_type=[
            jax.ShapeDtypeStruct((TOTAL, HID), jnp.float32),
            jax.ShapeDtypeStruct((TOTAL, HID), jnp.float32),
        ],
        scratch_types=[
            pltpu.VMEM((2, PER_W), jnp.int32),
            pltpu.VMEM((CHUNK, HID), jnp.float32),
            pltpu.VMEM((CHUNK, HID), jnp.float32),
            pltpu.SemaphoreType.DMA,
            pltpu.SemaphoreType.DMA,
        ],
    )
    eeg_flat, ecg_flat = run(eeg_table, ecg_table, eeg_idx, ecg_idx)
    return (eeg_flat.reshape(B, L, HID), ecg_flat.reshape(B, L, HID))


# speedup vs baseline: 3.5594x; 1.0731x over previous
import functools

import jax
import jax.numpy as jnp
from jax import lax
from jax.experimental import pallas as pl
from jax.experimental.pallas import tpu as pltpu
from jax.experimental.pallas import tpu_sc as plsc

B = 4096
L = 50
HID = 128
TOTAL = B * L
NW = 32
PER_W = TOTAL // NW
CHUNK = 128
NCHUNK = PER_W // CHUNK
NBUF = 5
AHEAD = 3


def _body(eeg_tab, ecg_tab, eeg_idx, ecg_idx, eeg_out, ecg_out,
          idx_v, *ring):
    bufs = ring[:NBUF]
    gsem = ring[NBUF:2 * NBUF]
    ssem = ring[2 * NBUF:]
    wid = lax.axis_index("c") * 16 + lax.axis_index("s")
    row_base = wid * PER_W

    for (tab, idx_hbm, out_hbm, ivm) in (
        (eeg_tab, eeg_idx, eeg_out, 0),
        (ecg_tab, ecg_idx, ecg_out, 1),
    ):
        pltpu.sync_copy(idx_hbm.at[pl.ds(row_base, PER_W)], idx_v.at[ivm])

        def gather(c, b):
            return pltpu.make_async_copy(
                tab.at[idx_v.at[ivm, pl.ds(c * CHUNK, CHUNK)]],
                bufs[b], gsem[b])

        def store(c, b):
            return pltpu.make_async_copy(
                bufs[b], out_hbm.at[pl.ds(row_base + c * CHUNK, CHUNK)],
                ssem[b])

        for c in range(AHEAD):
            gather(c, c).start()

        def turn(jj, _):
            for b in range(NBUF):
                cj = jj * NBUF + b
                @pl.when(cj >= NBUF - AHEAD)
                def _drain():
                    store(cj - (NBUF - AHEAD), (b + AHEAD) % NBUF).wait()

                @pl.when(cj + AHEAD < NCHUNK)
                def _fire():
                    gather(cj + AHEAD, (b + AHEAD) % NBUF).start()

                gather(cj, b).wait()
                store(cj, b).start()
            return _

        lax.fori_loop(0, NCHUNK // NBUF, turn, None)

        for c in range(NCHUNK - (NBUF - AHEAD), NCHUNK):
            store(c, c % NBUF).wait()


@functools.partial(jax.jit, static_argnums=())
def kernel(eeg_input_ids, ecg_input_ids, eeg_table, ecg_table):
    eeg_idx = eeg_input_ids.reshape(TOTAL).astype(jnp.int32)
    ecg_idx = ecg_input_ids.reshape(TOTAL).astype(jnp.int32)

    mesh = plsc.VectorSubcoreMesh(core_axis_name="c", subcore_axis_name="s")
    run = pl.kernel(
        _body,
        mesh=mesh,
        out_type=[
            jax.ShapeDtypeStruct((TOTAL, HID), jnp.float32),
            jax.ShapeDtypeStruct((TOTAL, HID), jnp.float32),
        ],
        scratch_types=(
            [pltpu.VMEM((2, PER_W), jnp.int32)]
            + [pltpu.VMEM((CHUNK, HID), jnp.float32)] * NBUF
            + [pltpu.SemaphoreType.DMA] * (2 * NBUF)
        ),
    )
    eeg_flat, ecg_flat = run(eeg_table, ecg_table, eeg_idx, ecg_idx)
    return (eeg_flat.reshape(B, L, HID), ecg_flat.reshape(B, L, HID))


# speedup vs baseline: 5.9847x; 1.6814x over previous
import functools

import jax
import jax.numpy as jnp
from jax import lax
from jax.experimental import pallas as pl
from jax.experimental.pallas import tpu as pltpu
from jax.experimental.pallas import tpu_sc as plsc

B = 4096
L = 50
HID = 128
NW = 32
ROWS_W = B // NW
G = 4
NGRP = ROWS_W // G
NBUF = 4
AHEAD = 2


def _body(eeg_tab, ecg_tab, eeg_idx, ecg_idx, eeg_out, ecg_out,
          idx_v, *ring):
    bufs = ring[:NBUF]
    gsem = ring[NBUF:2 * NBUF]
    ssem = ring[2 * NBUF:]
    wid = lax.axis_index("c") * 16 + lax.axis_index("s")
    row_base = wid * ROWS_W

    for (tab, idx_hbm, out_hbm) in (
        (eeg_tab, eeg_idx, eeg_out),
        (ecg_tab, ecg_idx, ecg_out),
    ):
        pltpu.sync_copy(idx_hbm.at[pl.ds(row_base, ROWS_W)], idx_v)

        def row_gathers(g, b):
            return [pltpu.make_async_copy(
                        tab.at[idx_v.at[g * G + r]],
                        bufs[b].at[r], gsem[b])
                    for r in range(G)]

        def gather_start(g, b):
            for d in row_gathers(g, b):
                d.start()

        def gather_wait(g, b):
            for d in row_gathers(g, b):
                d.wait()

        def store(g, b):
            return pltpu.make_async_copy(
                bufs[b], out_hbm.at[pl.ds(row_base + g * G, G)], ssem[b])

        for g in range(AHEAD):
            gather_start(g, g)

        def turn(jj, _):
            for b in range(NBUF):
                gj = jj * NBUF + b
                @pl.when(gj >= NBUF - AHEAD)
                def _drain():
                    store(gj - (NBUF - AHEAD), (b + AHEAD) % NBUF).wait()

                @pl.when(gj + AHEAD < NGRP)
                def _fire():
                    gather_start(gj + AHEAD, (b + AHEAD) % NBUF)

                gather_wait(gj, b)
                store(gj, b).start()
            return _

        lax.fori_loop(0, NGRP // NBUF, turn, None)

        for g in range(NGRP - (NBUF - AHEAD), NGRP):
            store(g, g % NBUF).wait()


@functools.partial(jax.jit, static_argnums=())
def kernel(eeg_input_ids, ecg_input_ids, eeg_table, ecg_table):
    eeg_idx = eeg_input_ids.astype(jnp.int32)
    ecg_idx = ecg_input_ids.astype(jnp.int32)

    mesh = plsc.VectorSubcoreMesh(core_axis_name="c", subcore_axis_name="s")
    run = pl.kernel(
        _body,
        mesh=mesh,
        out_type=[
            jax.ShapeDtypeStruct((B, L, HID), jnp.float32),
            jax.ShapeDtypeStruct((B, L, HID), jnp.float32),
        ],
        scratch_types=(
            [pltpu.VMEM((ROWS_W, L), jnp.int32)]
            + [pltpu.VMEM((G, L, HID), jnp.float32)] * NBUF
            + [pltpu.SemaphoreType.DMA] * (2 * NBUF)
        ),
    )
    return tuple(run(eeg_table, ecg_table, eeg_idx, ecg_idx))


# speedup vs baseline: 11.4231x; 1.9087x over previous
import functools

import jax
import jax.numpy as jnp
from jax import lax
from jax.experimental import pallas as pl
from jax.experimental.pallas import tpu as pltpu
from jax.experimental.pallas import tpu_sc as plsc

B = 4096
L = 50
HID = 128
NW = 32
ROWS_W = B // NW
NBUF = 5
AHEAD = 3


def _body(eeg_tab, ecg_tab, eeg_idx, ecg_idx, eeg_out, ecg_out,
          idx_v, *ring):
    bufs = ring[:NBUF]
    gsem = ring[NBUF:2 * NBUF]
    ssem = ring[2 * NBUF:]
    wid = lax.axis_index("c") * 16 + lax.axis_index("s")
    row_base = wid * ROWS_W

    for (tab, idx_hbm, out_hbm) in (
        (eeg_tab, eeg_idx, eeg_out),
        (ecg_tab, ecg_idx, ecg_out),
    ):
        pltpu.sync_copy(idx_hbm.at[:, pl.ds(row_base, ROWS_W)], idx_v)

        def gather(l, b):
            return pltpu.make_async_copy(
                tab.at[idx_v.at[l]], bufs[b], gsem[b])

        def store(l, b):
            return pltpu.make_async_copy(
                bufs[b], out_hbm.at[l, pl.ds(row_base, ROWS_W)], ssem[b])

        for l in range(AHEAD):
            gather(l, l).start()

        def turn(jj, _):
            for b in range(NBUF):
                lj = jj * NBUF + b
                @pl.when(lj >= NBUF - AHEAD)
                def _drain():
                    store(lj - (NBUF - AHEAD), (b + AHEAD) % NBUF).wait()

                @pl.when(lj + AHEAD < L)
                def _fire():
                    gather(lj + AHEAD, (b + AHEAD) % NBUF).start()

                gather(lj, b).wait()
                store(lj, b).start()
            return _

        lax.fori_loop(0, L // NBUF, turn, None)

        for l in range(L - (NBUF - AHEAD), L):
            store(l, l % NBUF).wait()


@functools.partial(jax.jit, static_argnums=())
def kernel(eeg_input_ids, ecg_input_ids, eeg_table, ecg_table):
    eeg_idx = eeg_input_ids.astype(jnp.int32).T
    ecg_idx = ecg_input_ids.astype(jnp.int32).T

    mesh = plsc.VectorSubcoreMesh(core_axis_name="c", subcore_axis_name="s")
    run = pl.kernel(
        _body,
        mesh=mesh,
        out_type=[
            jax.ShapeDtypeStruct((L, B, HID), jnp.float32),
            jax.ShapeDtypeStruct((L, B, HID), jnp.float32),
        ],
        scratch_types=(
            [pltpu.VMEM((L, ROWS_W), jnp.int32)]
            + [pltpu.VMEM((ROWS_W, HID), jnp.float32)] * NBUF
            + [pltpu.SemaphoreType.DMA] * (2 * NBUF)
        ),
    )
    eeg_t, ecg_t = run(eeg_table, ecg_table, eeg_idx, ecg_idx)
    return (eeg_t.transpose(1, 0, 2), ecg_t.transpose(1, 0, 2))
